# baseline (speedup 1.0000x reference)
import jax
import jax.numpy as jnp
from jax.experimental import pallas as pl
from jax.experimental.pallas import tpu as pltpu

B, L, O, DM = 8, 48, 100, 128


def _fused_step(x_ref, dis_ref, w1_ref, b1_ref, w2_ref, b2_ref, out_ref):
    x = x_ref[0]
    xt = x.T

    w1 = w1_ref[...]
    y = (jnp.dot(x, w1[:O], preferred_element_type=jnp.float32)
         + jnp.dot(xt, w1[O:], preferred_element_type=jnp.float32)
         + b1_ref[0])

    dis = dis_ref[...]
    sd = jnp.sqrt(dis)
    dis_w = jnp.where(dis <= 2.0, sd, 0.0) / jnp.sum(sd, axis=1, keepdims=True)
    f_sum = jnp.dot(dis_w, y, preferred_element_type=jnp.float32)
    w2 = w2_ref[...]
    b2 = b2_ref[0]
    geo = jnp.dot(y + f_sum, w2, preferred_element_type=jnp.float32) + b2

    td = jnp.sum(x, axis=0, keepdims=True) + jnp.sum(xt, axis=0, keepdims=True)
    tdn = td / jnp.sum(td)
    mask = ((x > 0) | (xt > 0)).astype(jnp.float32)
    deg_w = mask * tdn
    b_sum = jnp.dot(deg_w, y, preferred_element_type=jnp.float32)
    sem = jnp.dot(y + b_sum, w2, preferred_element_type=jnp.float32) + b2

    out_ref[0] = jnp.concatenate([geo, sem], axis=-1)


def kernel(X, dis_matrix, W1, b1, W2, b2):
    Bx, Lx, Ox, _ = X.shape
    n = Bx * Lx
    Xr = X.reshape(n, Ox, Ox)
    out = pl.pallas_call(
        _fused_step,
        grid=(n,),
        in_specs=[
            pl.BlockSpec((1, Ox, Ox), lambda i: (i, 0, 0)),
            pl.BlockSpec((Ox, Ox), lambda i: (0, 0)),
            pl.BlockSpec((2 * Ox, DM), lambda i: (0, 0)),
            pl.BlockSpec((1, DM), lambda i: (0, 0)),
            pl.BlockSpec((DM, DM), lambda i: (0, 0)),
            pl.BlockSpec((1, DM), lambda i: (0, 0)),
        ],
        out_specs=pl.BlockSpec((1, Ox, 2 * DM), lambda i: (i, 0, 0)),
        out_shape=jax.ShapeDtypeStruct((n, Ox, 2 * DM), jnp.float32),
    )(Xr, dis_matrix, W1, b1.reshape(1, DM), W2, b2.reshape(1, DM))
    return out.reshape(Bx, Lx, Ox, 2 * DM)


# speedup vs baseline: 1.4482x; 1.4482x over previous
import jax
import jax.numpy as jnp
from jax.experimental import pallas as pl
from jax.experimental.pallas import tpu as pltpu

B, L, O, DM = 8, 48, 100, 128
OP = 112
LT = 4


def _fused_step(x_ref, dis_ref, w1_ref, b1_ref, w2_ref, b2_ref, out_ref,
                disw_ref):
    @pl.when(pl.program_id(0) == 0)
    def _init():
        dis = dis_ref[...]
        sd = jnp.sqrt(dis)
        denom = jnp.sum(sd, axis=1, keepdims=True)
        denom = jnp.where(denom == 0.0, 1.0, denom)
        dw = jnp.where(dis <= 2.0, sd, 0.0) / denom
        disw_ref[...] = dw.astype(jnp.bfloat16)

    w1 = w1_ref[...]
    w2 = w2_ref[...]
    b1v = b1_ref[0]
    b2v = b2_ref[0]
    disw = disw_ref[...]

    for t in range(LT):
        x = x_ref[t]
        xt = x.T
        x16 = x.astype(jnp.bfloat16)
        xt16 = xt.astype(jnp.bfloat16)
        y = (jnp.dot(x16, w1[:OP], preferred_element_type=jnp.float32)
             + jnp.dot(xt16, w1[OP:], preferred_element_type=jnp.float32)
             + b1v)
        y16 = y.astype(jnp.bfloat16)

        td = (jnp.sum(x, axis=0, keepdims=True)
              + jnp.sum(xt, axis=0, keepdims=True))
        tdn = (td / jnp.sum(td)).astype(jnp.bfloat16)
        deg_w = jnp.where((x16 > 0) | (xt16 > 0),
                          jnp.broadcast_to(tdn, (OP, OP)),
                          jnp.bfloat16(0))

        lhs = jnp.concatenate([disw, deg_w], axis=0)
        agg = jnp.dot(lhs, y16, preferred_element_type=jnp.float32)
        f_sum = agg[:OP]
        b_sum = agg[OP:]

        geo = jnp.dot((y + f_sum).astype(jnp.bfloat16), w2,
                      preferred_element_type=jnp.float32) + b2v
        sem = jnp.dot((y + b_sum).astype(jnp.bfloat16), w2,
                      preferred_element_type=jnp.float32) + b2v
        out_ref[t] = jnp.concatenate([geo[:O], sem[:O]], axis=-1)


def kernel(X, dis_matrix, W1, b1, W2, b2):
    Bx, Lx, Ox, _ = X.shape
    n = Bx * Lx
    pad = OP - Ox
    Xp = jnp.pad(X, ((0, 0), (0, 0), (0, pad), (0, pad))).reshape(n, OP, OP)
    disp = jnp.pad(dis_matrix, ((0, pad), (0, pad)))
    W1p = jnp.concatenate([jnp.pad(W1[:Ox], ((0, pad), (0, 0))),
                           jnp.pad(W1[Ox:], ((0, pad), (0, 0)))],
                          axis=0).astype(jnp.bfloat16)
    W2b = W2.astype(jnp.bfloat16)

    out = pl.pallas_call(
        _fused_step,
        grid=(n // LT,),
        in_specs=[
            pl.BlockSpec((LT, OP, OP), lambda i: (i, 0, 0)),
            pl.BlockSpec((OP, OP), lambda i: (0, 0)),
            pl.BlockSpec((2 * OP, DM), lambda i: (0, 0)),
            pl.BlockSpec((1, DM), lambda i: (0, 0)),
            pl.BlockSpec((DM, DM), lambda i: (0, 0)),
            pl.BlockSpec((1, DM), lambda i: (0, 0)),
        ],
        out_specs=pl.BlockSpec((LT, Ox, 2 * DM), lambda i: (i, 0, 0)),
        out_shape=jax.ShapeDtypeStruct((n, Ox, 2 * DM), jnp.float32),
        scratch_shapes=[pltpu.VMEM((OP, OP), jnp.bfloat16)],
    )(Xp, disp, W1p, b1.reshape(1, DM), W2b, b2.reshape(1, DM))
    return out.reshape(Bx, Lx, Ox, 2 * DM)


# speedup vs baseline: 2.4028x; 1.6591x over previous
import jax
import jax.numpy as jnp
from jax.experimental import pallas as pl
from jax.experimental.pallas import tpu as pltpu

B, L, O, DM = 8, 48, 100, 128
OP = 112
LT = 8


def _fused_step(x_ref, dis_ref, w1_ref, b1_ref, w2_ref, b2_ref, out_ref,
                disw_ref):
    @pl.when(pl.program_id(0) == 0)
    def _init():
        dis = dis_ref[...]
        sd = jnp.sqrt(dis)
        denom = jnp.sum(sd, axis=1, keepdims=True)
        denom = jnp.where(denom == 0.0, 1.0, denom)
        dw = jnp.where(dis <= 2.0, sd, 0.0) / denom
        disw_ref[...] = dw.astype(jnp.bfloat16)

    w1 = w1_ref[...]
    w2 = w2_ref[...]
    b1v = b1_ref[0]
    b2v = b2_ref[0]
    disw = disw_ref[...]
    f32 = jnp.float32
    bf16 = jnp.bfloat16

    x3 = x_ref[...]
    xts = [x3[t].T for t in range(LT)]
    xtall = jnp.concatenate(xts, axis=0)
    xall = x3.reshape(LT * OP, OP)
    x16 = xall.astype(bf16)
    xt16 = xtall.astype(bf16)

    yall = (jnp.dot(x16, w1[:OP], preferred_element_type=f32)
            + jnp.dot(xt16, w1[OP:], preferred_element_type=f32)
            + b1v)
    y16 = yall.astype(bf16)

    td = (jnp.sum(x3, axis=1)
          + jnp.sum(xtall.reshape(LT, OP, OP), axis=1))
    tdn = (td / jnp.sum(td, axis=1, keepdims=True)).astype(bf16)

    ycat = jnp.concatenate([y16[t * OP:(t + 1) * OP] for t in range(LT)],
                           axis=1)
    fall = jnp.dot(disw, ycat, preferred_element_type=f32)

    geo_in = []
    sem_in = []
    for t in range(LT):
        sl = slice(t * OP, (t + 1) * OP)
        yt = yall[sl]
        deg_w = jnp.where((x16[sl] > 0) | (xt16[sl] > 0),
                          jnp.broadcast_to(tdn[t:t + 1], (OP, OP)),
                          bf16(0))
        bt = jnp.dot(deg_w, y16[sl], preferred_element_type=f32)
        geo_in.append((yt + fall[:, t * DM:(t + 1) * DM]).astype(bf16))
        sem_in.append((yt + bt).astype(bf16))

    geo_all = jnp.dot(jnp.concatenate(geo_in, axis=0), w2,
                      preferred_element_type=f32) + b2v
    sem_all = jnp.dot(jnp.concatenate(sem_in, axis=0), w2,
                      preferred_element_type=f32) + b2v
    for t in range(LT):
        out_ref[t] = jnp.concatenate([geo_all[t * OP:t * OP + O],
                                      sem_all[t * OP:t * OP + O]], axis=-1)


def kernel(X, dis_matrix, W1, b1, W2, b2):
    Bx, Lx, Ox, _ = X.shape
    n = Bx * Lx
    pad = OP - Ox
    Xp = jnp.pad(X, ((0, 0), (0, 0), (0, pad), (0, pad))).reshape(n, OP, OP)
    disp = jnp.pad(dis_matrix, ((0, pad), (0, pad)))
    W1p = jnp.concatenate([jnp.pad(W1[:Ox], ((0, pad), (0, 0))),
                           jnp.pad(W1[Ox:], ((0, pad), (0, 0)))],
                          axis=0).astype(jnp.bfloat16)
    W2b = W2.astype(jnp.bfloat16)

    out = pl.pallas_call(
        _fused_step,
        grid=(n // LT,),
        in_specs=[
            pl.BlockSpec((LT, OP, OP), lambda i: (i, 0, 0)),
            pl.BlockSpec((OP, OP), lambda i: (0, 0)),
            pl.BlockSpec((2 * OP, DM), lambda i: (0, 0)),
            pl.BlockSpec((1, DM), lambda i: (0, 0)),
            pl.BlockSpec((DM, DM), lambda i: (0, 0)),
            pl.BlockSpec((1, DM), lambda i: (0, 0)),
        ],
        out_specs=pl.BlockSpec((LT, Ox, 2 * DM), lambda i: (i, 0, 0)),
        out_shape=jax.ShapeDtypeStruct((n, Ox, 2 * DM), jnp.float32),
        scratch_shapes=[pltpu.VMEM((OP, OP), jnp.bfloat16)],
    )(Xp, disp, W1p, b1.reshape(1, DM), W2b, b2.reshape(1, DM))
    return out.reshape(Bx, Lx, Ox, 2 * DM)
